# baseline (speedup 1.0000x reference)
import functools

import jax
import jax.numpy as jnp
from jax import lax
from jax.experimental import pallas as pl
from jax.experimental.pallas import tpu as pltpu
from jax.experimental.pallas import tpu_sc as plsc

N = 10000
E = 320000
D = 128
H = 64
DE = 4
G = 256
NS = 16
CH = 128
NBUF = 2
NCHUNK = 160
EPS = CH * NCHUNK
EP = EPS * NS
NPAD = 10112
ZR = NPAD // NS

R = 2000
TCGRID = 5


def _sc_layer_body(xlo, xhi, srcu, dstu, srcv, dstv, elo, ehi, zeros,
                   aggu_lo, aggu_hi, aggv_lo, aggv_hi,
                   srcbuf, dstall, erows, rowsbuf,
                   acc, semi, semg, sems, semsrc):
    c = lax.axis_index("c")
    s = lax.axis_index("s")

    def run(x_ref, e_ref, out_u, out_v):
        zsl = pl.ds(s * ZR, ZR)
        pltpu.sync_copy(zeros.at[zsl], acc.at[zsl])
        plsc.subcore_barrier()

        def compute_group(par):
            def chunk(bb, carry0):
                rb = par * NBUF + bb

                def quad(qq, carry2):
                    for t in range(4):
                        i = qq * 4 + t
                        for v in range(H // 16):
                            sl = pl.ds(v * 16, 16)
                            rowsbuf[rb, i, sl] = jnp.maximum(
                                rowsbuf[rb, i, sl] + erows[rb, i, sl], 0.0)
                    return carry2

                lax.fori_loop(0, CH // 4, quad, 0)
                return carry0

            lax.fori_loop(0, NBUF, chunk, 0)

        def wait_rows(sem_slot, rb):
            pltpu.make_async_copy(x_ref.at[pl.ds(0, CH)], rowsbuf.at[rb],
                                  sem_slot).wait()

        def wait_e(b, slot):
            pltpu.make_async_copy(e_ref.at[pl.ds(0, CH)], erows.at[slot],
                                  semi.at[b]).wait()

        NGRP = NCHUNK // NBUF

        def side(src_idx, dst_idx, is_u):
            csl = pl.ds(s * NCHUNK, NCHUNK)
            pltpu.sync_copy(dst_idx.at[csl], dstall)
            abase = s * EPS
            rbase = s * NCHUNK

            def wait_src(b, slot):
                pltpu.make_async_copy(src_idx.at[0], srcbuf.at[slot],
                                      semsrc.at[b]).wait()

            for b in range(NBUF):
                pltpu.async_copy(src_idx.at[rbase + b], srcbuf.at[b],
                                 semsrc.at[b])
                pltpu.async_copy(src_idx.at[rbase + NBUF + b],
                                 srcbuf.at[NBUF + b], semsrc.at[b])
                if is_u:
                    pltpu.async_copy(e_ref.at[pl.ds(abase + b * CH, CH)],
                                     erows.at[b], semi.at[b])
            for b in range(NBUF):
                wait_src(b, b)
                pltpu.async_copy(x_ref.at[srcbuf.at[b]], rowsbuf.at[b],
                                 semg.at[b])

            def outer(og2, carry):
                for par in range(2):
                    og = og2 * 2 + par
                    pbase = par * NBUF
                    obase = (1 - par) * NBUF
                    jbase = og * NBUF
                    for b in range(NBUF):
                        wait_rows(semg.at[b], pbase + b)
                    if is_u:
                        for b in range(NBUF):
                            wait_e(b, pbase + b)

                    @pl.when(og + 1 < NGRP)
                    def _():
                        if is_u:
                            for b in range(NBUF):
                                pltpu.async_copy(
                                    e_ref.at[pl.ds(abase + (jbase + NBUF + b) * CH, CH)],
                                    erows.at[obase + b], semi.at[b])

                        @pl.when(og >= 1)
                        def _():
                            for b in range(NBUF):
                                wait_rows(sems.at[b], obase + b)

                        for b in range(NBUF):
                            wait_src(b, obase + b)
                            pltpu.async_copy(x_ref.at[srcbuf.at[obase + b]],
                                             rowsbuf.at[obase + b], semg.at[b])

                        @pl.when(og + 2 < NGRP)
                        def _():
                            for b in range(NBUF):
                                pltpu.async_copy(
                                    src_idx.at[rbase + jbase + 2 * NBUF + b],
                                    srcbuf.at[pbase + b], semsrc.at[b])

                    if is_u:
                        compute_group(par)

                    for b in range(NBUF):
                        pltpu.async_copy(rowsbuf.at[pbase + b],
                                         acc.at[dstall.at[jbase + b]],
                                         sems.at[b], add=True)
                return carry

            lax.fori_loop(0, NGRP // 2, outer, 0)

            for par in range(2):
                for b in range(NBUF):
                    wait_rows(sems.at[b], par * NBUF + b)

        side(srcu, dstu, True)
        plsc.subcore_barrier()
        pltpu.sync_copy(acc.at[zsl], out_u.at[zsl])
        pltpu.sync_copy(zeros.at[zsl], acc.at[zsl])
        plsc.subcore_barrier()
        side(srcv, dstv, False)
        plsc.subcore_barrier()
        pltpu.sync_copy(acc.at[zsl], out_v.at[zsl])

    @pl.when(c == 0)
    def _():
        run(xlo, elo, aggu_lo, aggv_lo)

    @pl.when(c == 1)
    def _():
        run(xhi, ehi, aggu_hi, aggv_hi)


def _make_sc_layer():
    mesh = plsc.VectorSubcoreMesh(core_axis_name="c", subcore_axis_name="s")
    f32 = jnp.float32
    return pl.kernel(
        _sc_layer_body,
        out_type=[jax.ShapeDtypeStruct((NPAD, H), f32) for _ in range(4)],
        mesh=mesh,
        scratch_types=[
            pltpu.VMEM((2 * NBUF, CH), jnp.int32),
            pltpu.VMEM((NCHUNK, CH), jnp.int32),
            pltpu.VMEM((2 * NBUF, CH, H), f32),
            pltpu.VMEM((2 * NBUF, CH, H), f32),
            pltpu.VMEM_SHARED((NPAD, H), f32),
            pltpu.SemaphoreType.DMA((NBUF,)),
            pltpu.SemaphoreType.DMA((NBUF,)),
            pltpu.SemaphoreType.DMA((NBUF,)),
            pltpu.SemaphoreType.DMA((NBUF,)),
        ],
        compiler_params=pltpu.CompilerParams(use_tc_tiling_on_sc=False),
    )


EB = 8192


def _e_body(attr_ref, we_ref, be_ref, lo_ref, hi_ref):
    e = jnp.dot(attr_ref[...], we_ref[0], preferred_element_type=jnp.float32)
    e = e + be_ref[0]
    lo_ref[0] = e[:, :H]
    hi_ref[0] = e[:, H:]


def _e_call(attr_p, Wes, bes):
    ngrid = EP // EB
    return pl.pallas_call(
        _e_body,
        grid=(3, ngrid),
        in_specs=[
            pl.BlockSpec((EB, DE), lambda l, i: (i, 0)),
            pl.BlockSpec((1, DE, D), lambda l, i: (l, 0, 0)),
            pl.BlockSpec((1, 1, D), lambda l, i: (l, 0, 0)),
        ],
        out_specs=[
            pl.BlockSpec((1, EB, H), lambda l, i: (l, i, 0)),
            pl.BlockSpec((1, EB, H), lambda l, i: (l, i, 0)),
        ],
        out_shape=[jax.ShapeDtypeStruct((3, EP, H), jnp.float32)] * 2,
    )(attr_p, Wes, bes)


def _atom_body(x_ref, w_ref, b_ref, lo_ref, hi_ref):
    y = jnp.dot(x_ref[...], w_ref[...], preferred_element_type=jnp.float32)
    y = y + b_ref[...]
    lo_ref[...] = y[:, :H]
    hi_ref[...] = y[:, H:]


def _atom_call(x, W_atom, b_atom):
    return pl.pallas_call(
        _atom_body,
        grid=(TCGRID,),
        in_specs=[
            pl.BlockSpec((R, D), lambda i: (i, 0)),
            pl.BlockSpec((D, D), lambda i: (0, 0)),
            pl.BlockSpec((1, D), lambda i: (0, 0)),
        ],
        out_specs=[
            pl.BlockSpec((R, H), lambda i: (i, 0)),
            pl.BlockSpec((R, H), lambda i: (i, 0)),
        ],
        out_shape=[jax.ShapeDtypeStruct((N, H), jnp.float32)] * 2,
    )(x, W_atom, b_atom)


def _tc_layer_body(aul_ref, auh_ref, avl_ref, avh_ref, xl_ref, xh_ref,
                   wu_ref, wv_ref, g_ref, b_ref, lo_ref, hi_ref):
    wu = wu_ref[...]
    wv = wv_ref[...]
    s = (jnp.dot(aul_ref[...], wu[:H, :], preferred_element_type=jnp.float32)
         + jnp.dot(auh_ref[...], wu[H:, :], preferred_element_type=jnp.float32)
         + jnp.dot(avl_ref[...], wv[:H, :], preferred_element_type=jnp.float32)
         + jnp.dot(avh_ref[...], wv[H:, :], preferred_element_type=jnp.float32))
    mu = jnp.mean(s, axis=-1, keepdims=True)
    d = s - mu
    var = jnp.mean(d * d, axis=-1, keepdims=True)
    h = jax.nn.relu(d * lax.rsqrt(var + 1e-5) * g_ref[...] + b_ref[...])
    lo_ref[...] = h[:, :H] + xl_ref[...]
    hi_ref[...] = h[:, H:] + xh_ref[...]


def _tc_layer_call(aul, auh, avl, avh, xl, xh, WuL, WvL, gamma, beta):
    half = pl.BlockSpec((R, H), lambda i: (i, 0))
    full = pl.BlockSpec((D, D), lambda i: (0, 0))
    vec = pl.BlockSpec((1, D), lambda i: (0, 0))
    return pl.pallas_call(
        _tc_layer_body,
        grid=(TCGRID,),
        in_specs=[half, half, half, half, half, half, full, full, vec, vec],
        out_specs=[half, half],
        out_shape=[jax.ShapeDtypeStruct((N, H), jnp.float32)] * 2,
    )(aul, auh, avl, avh, xl, xh, WuL, WvL, gamma, beta)


def _pool_body(xl_ref, xh_ref, ids_ref, w_ref, b_ref, out_ref):
    i = pl.program_id(0)
    w = w_ref[...]
    y = (jnp.dot(xl_ref[...], w[:H, :], preferred_element_type=jnp.float32)
         + jnp.dot(xh_ref[...], w[H:, :], preferred_element_type=jnp.float32))
    ids = ids_ref[...].reshape(1, R)
    onehot = (lax.broadcasted_iota(jnp.int32, (G, R), 0) == ids).astype(jnp.float32)
    part = jnp.dot(onehot, y, preferred_element_type=jnp.float32)

    @pl.when(i == 0)
    def _():
        out_ref[...] = part + jnp.broadcast_to(b_ref[...], (G, D))

    @pl.when(i > 0)
    def _():
        out_ref[...] = out_ref[...] + part


def _pool_call(xl, xh, ids3, W_pool, b_pool):
    return pl.pallas_call(
        _pool_body,
        grid=(TCGRID,),
        in_specs=[
            pl.BlockSpec((R, H), lambda i: (i, 0)),
            pl.BlockSpec((R, H), lambda i: (i, 0)),
            pl.BlockSpec((1, 1, R), lambda i: (i, 0, 0)),
            pl.BlockSpec((D, D), lambda i: (0, 0)),
            pl.BlockSpec((1, D), lambda i: (0, 0)),
        ],
        out_specs=pl.BlockSpec((G, D), lambda i: (0, 0)),
        out_shape=jax.ShapeDtypeStruct((G, D), jnp.float32),
    )(xl, xh, ids3, W_pool, b_pool)


def kernel(x, index_uL, index_vL, attr_uL, batch_ids, W_atom, b_atom,
           We0, be0, WuL0, WvL0, gamma0, beta0,
           We1, be1, WuL1, WvL1, gamma1, beta1,
           We2, be2, WuL2, WvL2, gamma2, beta2,
           W_pool, b_pool):
    i32 = jnp.int32
    f32 = jnp.float32
    pad = EP - E

    def pad_idx(a, val):
        return jnp.concatenate([a, jnp.full((pad,), val, i32)]).reshape(
            NS * NCHUNK, CH)

    dstu = pad_idx(index_uL[0], N)
    srcu = pad_idx(index_uL[1], 0)
    dstv = pad_idx(index_vL[0], N)
    srcv = pad_idx(index_vL[1], 0)
    attr_p = jnp.concatenate([attr_uL, jnp.zeros((pad, DE), f32)])
    zeros = jnp.zeros((NPAD, H), f32)
    ids3 = batch_ids.reshape(TCGRID, 1, R)

    Wes = jnp.stack([We0, We1, We2])
    bes = jnp.stack([be0.reshape(1, D), be1.reshape(1, D), be2.reshape(1, D)])
    els, ehs = _e_call(attr_p, Wes, bes)

    xl, xh = _atom_call(x, W_atom, b_atom.reshape(1, D))

    sc_layer = _make_sc_layer()
    Wus = jnp.stack([WuL0, WuL1, WuL2])
    Wvs = jnp.stack([WvL0, WvL1, WvL2])
    gs = jnp.stack([gamma0.reshape(1, D), gamma1.reshape(1, D), gamma2.reshape(1, D)])
    bs = jnp.stack([beta0.reshape(1, D), beta1.reshape(1, D), beta2.reshape(1, D)])

    def step(carry, ws):
        cxl, cxh = carry
        el, eh, WuL, WvL, g, b = ws
        aul, auh, avl, avh = sc_layer(
            cxl, cxh, srcu, dstu, srcv, dstv, el, eh, zeros)
        nxl, nxh = _tc_layer_call(aul, auh, avl, avh, cxl, cxh, WuL, WvL, g, b)
        return (nxl, nxh), None

    (xl, xh), _ = lax.scan(step, (xl, xh), (els, ehs, Wus, Wvs, gs, bs))

    return _pool_call(xl, xh, ids3, W_pool, b_pool.reshape(1, D))


# speedup vs baseline: 1.1963x; 1.1963x over previous
import functools

import jax
import jax.numpy as jnp
from jax import lax
from jax.experimental import pallas as pl
from jax.experimental.pallas import tpu as pltpu
from jax.experimental.pallas import tpu_sc as plsc

N = 10000
E = 320000
D = 128
H = 64
DE = 4
G = 256
NS = 16
CH = 128
NBUF = 2
NCHUNK = 160
EPS = CH * NCHUNK
EP = EPS * NS
NPAD = 10112
ZR = NPAD // NS

R = 2000
TCGRID = 5


def _sc_layer_body(xlo, xhi, srcu, dstu, srcv, dstv, elo, ehi, zeros,
                   aggu_lo, aggu_hi, aggv_lo, aggv_hi,
                   srcbuf, dstall, erows, rowsbuf,
                   acc, semi, semg, sems, semsrc):
    c = lax.axis_index("c")
    s = lax.axis_index("s")

    def run(x_ref, e_ref, out_u, out_v):
        zsl = pl.ds(s * ZR, ZR)
        pltpu.sync_copy(zeros.at[zsl], acc.at[zsl])
        plsc.subcore_barrier()

        def compute_group(par):
            def chunk(bb, carry0):
                rb = par * NBUF + bb

                def quad(qq, carry2):
                    m = qq // 8
                    il0 = qq * 4 - m * 32
                    for t in range(4):
                        i = qq * 4 + t
                        il = il0 + t
                        for v in range(H // 16):
                            sl = pl.ds(v * 16, 16)
                            esl = pl.ds(il * H + v * 16, 16)
                            rowsbuf[rb, i, sl] = jnp.maximum(
                                rowsbuf[rb, i, sl] + erows[rb, m, esl], 0.0)
                    return carry2

                lax.fori_loop(0, CH // 4, quad, 0)
                return carry0

            lax.fori_loop(0, NBUF, chunk, 0)

        def wait_rows(sem_slot, rb):
            pltpu.make_async_copy(x_ref.at[pl.ds(0, CH)], rowsbuf.at[rb],
                                  sem_slot).wait()

        def wait_e(b, slot):
            pltpu.make_async_copy(e_ref.at[pl.ds(0, 4)], erows.at[slot],
                                  semi.at[b]).wait()

        NGRP = NCHUNK // NBUF

        def side(src_idx, dst_idx, is_u):
            csl = pl.ds(s * NCHUNK, NCHUNK)
            pltpu.sync_copy(dst_idx.at[csl], dstall)
            ebase = s * (EPS // 32)
            rbase = s * NCHUNK

            def wait_src(b, slot):
                pltpu.make_async_copy(src_idx.at[0], srcbuf.at[slot],
                                      semsrc.at[b]).wait()

            for b in range(NBUF):
                pltpu.async_copy(src_idx.at[rbase + b], srcbuf.at[b],
                                 semsrc.at[b])
                pltpu.async_copy(src_idx.at[rbase + NBUF + b],
                                 srcbuf.at[NBUF + b], semsrc.at[b])
                if is_u:
                    pltpu.async_copy(e_ref.at[pl.ds(ebase + b * 4, 4)],
                                     erows.at[b], semi.at[b])
            for b in range(NBUF):
                wait_src(b, b)
                pltpu.async_copy(x_ref.at[srcbuf.at[b]], rowsbuf.at[b],
                                 semg.at[b])

            def outer(og2, carry):
                for par in range(2):
                    og = og2 * 2 + par
                    pbase = par * NBUF
                    obase = (1 - par) * NBUF
                    jbase = og * NBUF
                    for b in range(NBUF):
                        wait_rows(semg.at[b], pbase + b)
                    if is_u:
                        for b in range(NBUF):
                            wait_e(b, pbase + b)

                    @pl.when(og + 1 < NGRP)
                    def _():
                        if is_u:
                            for b in range(NBUF):
                                pltpu.async_copy(
                                    e_ref.at[pl.ds(ebase + (jbase + NBUF + b) * 4, 4)],
                                    erows.at[obase + b], semi.at[b])

                        @pl.when(og >= 1)
                        def _():
                            for b in range(NBUF):
                                wait_rows(sems.at[b], obase + b)

                        for b in range(NBUF):
                            wait_src(b, obase + b)
                            pltpu.async_copy(x_ref.at[srcbuf.at[obase + b]],
                                             rowsbuf.at[obase + b], semg.at[b])

                        @pl.when(og + 2 < NGRP)
                        def _():
                            for b in range(NBUF):
                                pltpu.async_copy(
                                    src_idx.at[rbase + jbase + 2 * NBUF + b],
                                    srcbuf.at[pbase + b], semsrc.at[b])

                    if is_u:
                        compute_group(par)

                    for b in range(NBUF):
                        pltpu.async_copy(rowsbuf.at[pbase + b],
                                         acc.at[dstall.at[jbase + b]],
                                         sems.at[b], add=True)
                return carry

            lax.fori_loop(0, NGRP // 2, outer, 0)

            for par in range(2):
                for b in range(NBUF):
                    wait_rows(sems.at[b], par * NBUF + b)

        side(srcu, dstu, True)
        plsc.subcore_barrier()
        pltpu.sync_copy(acc.at[zsl], out_u.at[zsl])
        pltpu.sync_copy(zeros.at[zsl], acc.at[zsl])
        plsc.subcore_barrier()
        side(srcv, dstv, False)
        plsc.subcore_barrier()
        pltpu.sync_copy(acc.at[zsl], out_v.at[zsl])

    @pl.when(c == 0)
    def _():
        run(xlo, elo, aggu_lo, aggv_lo)

    @pl.when(c == 1)
    def _():
        run(xhi, ehi, aggu_hi, aggv_hi)


def _make_sc_layer():
    mesh = plsc.VectorSubcoreMesh(core_axis_name="c", subcore_axis_name="s")
    f32 = jnp.float32
    return pl.kernel(
        _sc_layer_body,
        out_type=[jax.ShapeDtypeStruct((NPAD, H), f32) for _ in range(4)],
        mesh=mesh,
        scratch_types=[
            pltpu.VMEM((2 * NBUF, CH), jnp.int32),
            pltpu.VMEM((NCHUNK, CH), jnp.int32),
            pltpu.VMEM((2 * NBUF, 4, 32 * H), f32),
            pltpu.VMEM((2 * NBUF, CH, H), f32),
            pltpu.VMEM_SHARED((NPAD, H), f32),
            pltpu.SemaphoreType.DMA((NBUF,)),
            pltpu.SemaphoreType.DMA((NBUF,)),
            pltpu.SemaphoreType.DMA((NBUF,)),
            pltpu.SemaphoreType.DMA((NBUF,)),
        ],
        compiler_params=pltpu.CompilerParams(use_tc_tiling_on_sc=False),
    )


M2 = EP // 32
EB2 = 256
EW = 32 * H


def _e_body(attr_ref, wlo_ref, whi_ref, lo_ref, hi_ref):
    a = attr_ref[...]
    lo_ref[0] = jnp.dot(a, wlo_ref[0], preferred_element_type=jnp.float32)
    hi_ref[0] = jnp.dot(a, whi_ref[0], preferred_element_type=jnp.float32)


def _e_call(attr2, Wlo, Whi):
    ngrid = M2 // EB2
    return pl.pallas_call(
        _e_body,
        grid=(3, ngrid),
        in_specs=[
            pl.BlockSpec((EB2, 256), lambda l, i: (i, 0)),
            pl.BlockSpec((1, 256, EW), lambda l, i: (l, 0, 0)),
            pl.BlockSpec((1, 256, EW), lambda l, i: (l, 0, 0)),
        ],
        out_specs=[
            pl.BlockSpec((1, EB2, EW), lambda l, i: (l, i, 0)),
            pl.BlockSpec((1, EB2, EW), lambda l, i: (l, i, 0)),
        ],
        out_shape=[jax.ShapeDtypeStruct((3, M2, EW), jnp.float32)] * 2,
    )(attr2, Wlo, Whi)


def _atom_body(x_ref, w_ref, b_ref, lo_ref, hi_ref):
    y = jnp.dot(x_ref[...], w_ref[...], preferred_element_type=jnp.float32)
    y = y + b_ref[...]
    lo_ref[...] = y[:, :H]
    hi_ref[...] = y[:, H:]


def _atom_call(x, W_atom, b_atom):
    return pl.pallas_call(
        _atom_body,
        grid=(TCGRID,),
        in_specs=[
            pl.BlockSpec((R, D), lambda i: (i, 0)),
            pl.BlockSpec((D, D), lambda i: (0, 0)),
            pl.BlockSpec((1, D), lambda i: (0, 0)),
        ],
        out_specs=[
            pl.BlockSpec((R, H), lambda i: (i, 0)),
            pl.BlockSpec((R, H), lambda i: (i, 0)),
        ],
        out_shape=[jax.ShapeDtypeStruct((N, H), jnp.float32)] * 2,
    )(x, W_atom, b_atom)


def _tc_layer_body(aul_ref, auh_ref, avl_ref, avh_ref, xl_ref, xh_ref,
                   wu_ref, wv_ref, g_ref, b_ref, lo_ref, hi_ref):
    wu = wu_ref[...]
    wv = wv_ref[...]
    s = (jnp.dot(aul_ref[...], wu[:H, :], preferred_element_type=jnp.float32)
         + jnp.dot(auh_ref[...], wu[H:, :], preferred_element_type=jnp.float32)
         + jnp.dot(avl_ref[...], wv[:H, :], preferred_element_type=jnp.float32)
         + jnp.dot(avh_ref[...], wv[H:, :], preferred_element_type=jnp.float32))
    mu = jnp.mean(s, axis=-1, keepdims=True)
    d = s - mu
    var = jnp.mean(d * d, axis=-1, keepdims=True)
    h = jax.nn.relu(d * lax.rsqrt(var + 1e-5) * g_ref[...] + b_ref[...])
    lo_ref[...] = h[:, :H] + xl_ref[...]
    hi_ref[...] = h[:, H:] + xh_ref[...]


def _tc_layer_call(aul, auh, avl, avh, xl, xh, WuL, WvL, gamma, beta):
    half = pl.BlockSpec((R, H), lambda i: (i, 0))
    full = pl.BlockSpec((D, D), lambda i: (0, 0))
    vec = pl.BlockSpec((1, D), lambda i: (0, 0))
    return pl.pallas_call(
        _tc_layer_body,
        grid=(TCGRID,),
        in_specs=[half, half, half, half, half, half, full, full, vec, vec],
        out_specs=[half, half],
        out_shape=[jax.ShapeDtypeStruct((N, H), jnp.float32)] * 2,
    )(aul, auh, avl, avh, xl, xh, WuL, WvL, gamma, beta)


def _pool_body(xl_ref, xh_ref, ids_ref, w_ref, b_ref, out_ref):
    i = pl.program_id(0)
    w = w_ref[...]
    y = (jnp.dot(xl_ref[...], w[:H, :], preferred_element_type=jnp.float32)
         + jnp.dot(xh_ref[...], w[H:, :], preferred_element_type=jnp.float32))
    ids = ids_ref[...].reshape(1, R)
    onehot = (lax.broadcasted_iota(jnp.int32, (G, R), 0) == ids).astype(jnp.float32)
    part = jnp.dot(onehot, y, preferred_element_type=jnp.float32)

    @pl.when(i == 0)
    def _():
        out_ref[...] = part + jnp.broadcast_to(b_ref[...], (G, D))

    @pl.when(i > 0)
    def _():
        out_ref[...] = out_ref[...] + part


def _pool_call(xl, xh, ids3, W_pool, b_pool):
    return pl.pallas_call(
        _pool_body,
        grid=(TCGRID,),
        in_specs=[
            pl.BlockSpec((R, H), lambda i: (i, 0)),
            pl.BlockSpec((R, H), lambda i: (i, 0)),
            pl.BlockSpec((1, 1, R), lambda i: (i, 0, 0)),
            pl.BlockSpec((D, D), lambda i: (0, 0)),
            pl.BlockSpec((1, D), lambda i: (0, 0)),
        ],
        out_specs=pl.BlockSpec((G, D), lambda i: (0, 0)),
        out_shape=jax.ShapeDtypeStruct((G, D), jnp.float32),
    )(xl, xh, ids3, W_pool, b_pool)


def kernel(x, index_uL, index_vL, attr_uL, batch_ids, W_atom, b_atom,
           We0, be0, WuL0, WvL0, gamma0, beta0,
           We1, be1, WuL1, WvL1, gamma1, beta1,
           We2, be2, WuL2, WvL2, gamma2, beta2,
           W_pool, b_pool):
    i32 = jnp.int32
    f32 = jnp.float32
    pad = EP - E

    def pad_idx(a, val):
        return jnp.concatenate([a, jnp.full((pad,), val, i32)]).reshape(
            NS * NCHUNK, CH)

    dstu = pad_idx(index_uL[0], N)
    srcu = pad_idx(index_uL[1], 0)
    dstv = pad_idx(index_vL[0], N)
    srcv = pad_idx(index_vL[1], 0)
    attr_p = jnp.concatenate([attr_uL, jnp.zeros((pad, DE), f32)])
    zeros = jnp.zeros((NPAD, H), f32)
    ids3 = batch_ids.reshape(TCGRID, 1, R)

    eye32 = jnp.eye(32, dtype=f32)
    attr2 = jnp.concatenate(
        [attr_p, jnp.ones((EP, 1), f32), jnp.zeros((EP, 3), f32)],
        axis=1).reshape(M2, 256)

    def wblk(We, be, lo):
        half = We[:, :H] if lo else We[:, H:]
        bhalf = (be[:H] if lo else be[H:]).reshape(1, H)
        blk = jnp.concatenate([half, bhalf, jnp.zeros((3, H), f32)])
        return jnp.kron(eye32, blk)

    Wlo = jnp.stack([wblk(We, be, True)
                     for We, be in ((We0, be0), (We1, be1), (We2, be2))])
    Whi = jnp.stack([wblk(We, be, False)
                     for We, be in ((We0, be0), (We1, be1), (We2, be2))])
    els, ehs = _e_call(attr2, Wlo, Whi)

    xl, xh = _atom_call(x, W_atom, b_atom.reshape(1, D))

    sc_layer = _make_sc_layer()
    Wus = jnp.stack([WuL0, WuL1, WuL2])
    Wvs = jnp.stack([WvL0, WvL1, WvL2])
    gs = jnp.stack([gamma0.reshape(1, D), gamma1.reshape(1, D), gamma2.reshape(1, D)])
    bs = jnp.stack([beta0.reshape(1, D), beta1.reshape(1, D), beta2.reshape(1, D)])

    def step(carry, ws):
        cxl, cxh = carry
        el, eh, WuL, WvL, g, b = ws
        aul, auh, avl, avh = sc_layer(
            cxl, cxh, srcu, dstu, srcv, dstv, el, eh, zeros)
        nxl, nxh = _tc_layer_call(aul, auh, avl, avh, cxl, cxh, WuL, WvL, g, b)
        return (nxl, nxh), None

    (xl, xh), _ = lax.scan(step, (xl, xh), (els, ehs, Wus, Wvs, gs, bs))

    return _pool_call(xl, xh, ids3, W_pool, b_pool.reshape(1, D))


# speedup vs baseline: 1.2692x; 1.0609x over previous
import functools

import jax
import jax.numpy as jnp
from jax import lax
from jax.experimental import pallas as pl
from jax.experimental.pallas import tpu as pltpu
from jax.experimental.pallas import tpu_sc as plsc

N = 10000
E = 320000
D = 128
H = 64
DE = 4
G = 256
NS = 16
CH = 128
NBUF = 4
NCHUNK = 160
EPS = CH * NCHUNK
EP = EPS * NS
NPAD = 10112
ZR = NPAD // NS

R = 2000
TCGRID = 5


def _sc_layer_body(xlo, xhi, srcu, dstu, srcv, dstv, attr,
                   welo, wehi, belo, behi, zeros,
                   aggu_lo, aggu_hi, aggv_lo, aggv_hi,
                   we_v, be_v, srcbuf, dstall, attrbuf, rowsbuf,
                   acc, semi, semg, sems, semsrc):
    c = lax.axis_index("c")
    s = lax.axis_index("s")

    def run(x_ref, we_ref, be_ref, out_u, out_v):
        zsl = pl.ds(s * ZR, ZR)
        pltpu.sync_copy(we_ref, we_v)
        pltpu.sync_copy(be_ref, be_v)
        pltpu.sync_copy(zeros.at[zsl], acc.at[zsl])
        plsc.subcore_barrier()

        def compute_group(par):
            def chunk(bb, carry0):
                rb = par * NBUF + bb

                def group(gg, carry2):
                    avs = [attrbuf[bb, k, pl.ds(gg * 16, 16)]
                           for k in range(DE)]
                    for t in range(16):
                        i = gg * 16 + t
                        for v in range(H // 16):
                            sl = pl.ds(v * 16, 16)
                            e = (be_v[sl]
                                 + avs[0][t] * we_v[0, sl]
                                 + avs[1][t] * we_v[1, sl]
                                 + avs[2][t] * we_v[2, sl]
                                 + avs[3][t] * we_v[3, sl])
                            rowsbuf[rb, i, sl] = jnp.maximum(
                                rowsbuf[rb, i, sl] + e, 0.0)
                    return carry2

                lax.fori_loop(0, CH // 16, group, 0)
                return carry0

            lax.fori_loop(0, NBUF, chunk, 0)

        def wait_rows(sem_slot, rb):
            pltpu.make_async_copy(x_ref.at[pl.ds(0, CH)], rowsbuf.at[rb],
                                  sem_slot).wait()

        def wait_attr(b, slot):
            pltpu.make_async_copy(attr.at[:, pl.ds(0, CH)], attrbuf.at[slot],
                                  semi.at[b]).wait()

        NGRP = NCHUNK // NBUF

        def side(src_idx, dst_idx, is_u):
            csl = pl.ds(s * NCHUNK, NCHUNK)
            pltpu.sync_copy(dst_idx.at[csl], dstall)
            abase = s * EPS
            rbase = s * NCHUNK

            def wait_src(b, slot):
                pltpu.make_async_copy(src_idx.at[0], srcbuf.at[slot],
                                      semsrc.at[b]).wait()

            for b in range(NBUF):
                pltpu.async_copy(src_idx.at[rbase + b], srcbuf.at[b],
                                 semsrc.at[b])
                pltpu.async_copy(src_idx.at[rbase + NBUF + b],
                                 srcbuf.at[NBUF + b], semsrc.at[b])
                if is_u:
                    pltpu.async_copy(attr.at[:, pl.ds(abase + b * CH, CH)],
                                     attrbuf.at[b], semi.at[b])
            for b in range(NBUF):
                wait_src(b, b)
                pltpu.async_copy(x_ref.at[srcbuf.at[b]], rowsbuf.at[b],
                                 semg.at[b])

            def outer(og2, carry):
                for par in range(2):
                    og = og2 * 2 + par
                    pbase = par * NBUF
                    obase = (1 - par) * NBUF
                    jbase = og * NBUF
                    for b in range(NBUF):
                        wait_rows(semg.at[b], pbase + b)
                    if is_u:
                        for b in range(NBUF):
                            wait_attr(b, b)

                    @pl.when(og + 1 < NGRP)
                    def _():
                        @pl.when(og >= 1)
                        def _():
                            for b in range(NBUF):
                                wait_rows(sems.at[b], obase + b)

                        for b in range(NBUF):
                            wait_src(b, obase + b)
                            pltpu.async_copy(x_ref.at[srcbuf.at[obase + b]],
                                             rowsbuf.at[obase + b], semg.at[b])

                        @pl.when(og + 2 < NGRP)
                        def _():
                            for b in range(NBUF):
                                pltpu.async_copy(
                                    src_idx.at[rbase + jbase + 2 * NBUF + b],
                                    srcbuf.at[pbase + b], semsrc.at[b])

                    if is_u:
                        compute_group(par)

                        @pl.when(og + 1 < NGRP)
                        def _():
                            for b in range(NBUF):
                                pltpu.async_copy(
                                    attr.at[:, pl.ds(abase + (jbase + NBUF + b) * CH, CH)],
                                    attrbuf.at[b], semi.at[b])

                    for b in range(NBUF):
                        pltpu.async_copy(rowsbuf.at[pbase + b],
                                         acc.at[dstall.at[jbase + b]],
                                         sems.at[b], add=True)
                return carry

            lax.fori_loop(0, NGRP // 2, outer, 0)

            for par in range(2):
                for b in range(NBUF):
                    wait_rows(sems.at[b], par * NBUF + b)

        side(srcu, dstu, True)
        plsc.subcore_barrier()
        pltpu.sync_copy(acc.at[zsl], out_u.at[zsl])
        pltpu.sync_copy(zeros.at[zsl], acc.at[zsl])
        plsc.subcore_barrier()
        side(srcv, dstv, False)
        plsc.subcore_barrier()
        pltpu.sync_copy(acc.at[zsl], out_v.at[zsl])

    @pl.when(c == 0)
    def _():
        run(xlo, welo, belo, aggu_lo, aggv_lo)

    @pl.when(c == 1)
    def _():
        run(xhi, wehi, behi, aggu_hi, aggv_hi)


def _make_sc_layer():
    mesh = plsc.VectorSubcoreMesh(core_axis_name="c", subcore_axis_name="s")
    f32 = jnp.float32
    return pl.kernel(
        _sc_layer_body,
        out_type=[jax.ShapeDtypeStruct((NPAD, H), f32) for _ in range(4)],
        mesh=mesh,
        scratch_types=[
            pltpu.VMEM((DE, H), f32),
            pltpu.VMEM((H,), f32),
            pltpu.VMEM((2 * NBUF, CH), jnp.int32),
            pltpu.VMEM((NCHUNK, CH), jnp.int32),
            pltpu.VMEM((NBUF, DE, CH), f32),
            pltpu.VMEM((2 * NBUF, CH, H), f32),
            pltpu.VMEM_SHARED((NPAD, H), f32),
            pltpu.SemaphoreType.DMA((NBUF,)),
            pltpu.SemaphoreType.DMA((NBUF,)),
            pltpu.SemaphoreType.DMA((NBUF,)),
            pltpu.SemaphoreType.DMA((NBUF,)),
        ],
        compiler_params=pltpu.CompilerParams(use_tc_tiling_on_sc=False),
    )


def _atom_body(x_ref, w_ref, b_ref, lo_ref, hi_ref):
    y = jnp.dot(x_ref[...], w_ref[...], preferred_element_type=jnp.float32)
    y = y + b_ref[...]
    lo_ref[...] = y[:, :H]
    hi_ref[...] = y[:, H:]


def _atom_call(x, W_atom, b_atom):
    return pl.pallas_call(
        _atom_body,
        grid=(TCGRID,),
        in_specs=[
            pl.BlockSpec((R, D), lambda i: (i, 0)),
            pl.BlockSpec((D, D), lambda i: (0, 0)),
            pl.BlockSpec((1, D), lambda i: (0, 0)),
        ],
        out_specs=[
            pl.BlockSpec((R, H), lambda i: (i, 0)),
            pl.BlockSpec((R, H), lambda i: (i, 0)),
        ],
        out_shape=[jax.ShapeDtypeStruct((N, H), jnp.float32)] * 2,
    )(x, W_atom, b_atom)


def _tc_layer_body(aul_ref, auh_ref, avl_ref, avh_ref, xl_ref, xh_ref,
                   wu_ref, wv_ref, g_ref, b_ref, lo_ref, hi_ref):
    wu = wu_ref[...]
    wv = wv_ref[...]
    s = (jnp.dot(aul_ref[...], wu[:H, :], preferred_element_type=jnp.float32)
         + jnp.dot(auh_ref[...], wu[H:, :], preferred_element_type=jnp.float32)
         + jnp.dot(avl_ref[...], wv[:H, :], preferred_element_type=jnp.float32)
         + jnp.dot(avh_ref[...], wv[H:, :], preferred_element_type=jnp.float32))
    mu = jnp.mean(s, axis=-1, keepdims=True)
    d = s - mu
    var = jnp.mean(d * d, axis=-1, keepdims=True)
    h = jax.nn.relu(d * lax.rsqrt(var + 1e-5) * g_ref[...] + b_ref[...])
    lo_ref[...] = h[:, :H] + xl_ref[...]
    hi_ref[...] = h[:, H:] + xh_ref[...]


def _tc_layer_call(aul, auh, avl, avh, xl, xh, WuL, WvL, gamma, beta):
    half = pl.BlockSpec((R, H), lambda i: (i, 0))
    full = pl.BlockSpec((D, D), lambda i: (0, 0))
    vec = pl.BlockSpec((1, D), lambda i: (0, 0))
    return pl.pallas_call(
        _tc_layer_body,
        grid=(TCGRID,),
        in_specs=[half, half, half, half, half, half, full, full, vec, vec],
        out_specs=[half, half],
        out_shape=[jax.ShapeDtypeStruct((N, H), jnp.float32)] * 2,
    )(aul, auh, avl, avh, xl, xh, WuL, WvL, gamma, beta)


def _pool_body(xl_ref, xh_ref, ids_ref, w_ref, b_ref, out_ref):
    i = pl.program_id(0)
    w = w_ref[...]
    y = (jnp.dot(xl_ref[...], w[:H, :], preferred_element_type=jnp.float32)
         + jnp.dot(xh_ref[...], w[H:, :], preferred_element_type=jnp.float32))
    ids = ids_ref[...].reshape(1, R)
    onehot = (lax.broadcasted_iota(jnp.int32, (G, R), 0) == ids).astype(jnp.float32)
    part = jnp.dot(onehot, y, preferred_element_type=jnp.float32)

    @pl.when(i == 0)
    def _():
        out_ref[...] = part + jnp.broadcast_to(b_ref[...], (G, D))

    @pl.when(i > 0)
    def _():
        out_ref[...] = out_ref[...] + part


def _pool_call(xl, xh, ids3, W_pool, b_pool):
    return pl.pallas_call(
        _pool_body,
        grid=(TCGRID,),
        in_specs=[
            pl.BlockSpec((R, H), lambda i: (i, 0)),
            pl.BlockSpec((R, H), lambda i: (i, 0)),
            pl.BlockSpec((1, 1, R), lambda i: (i, 0, 0)),
            pl.BlockSpec((D, D), lambda i: (0, 0)),
            pl.BlockSpec((1, D), lambda i: (0, 0)),
        ],
        out_specs=pl.BlockSpec((G, D), lambda i: (0, 0)),
        out_shape=jax.ShapeDtypeStruct((G, D), jnp.float32),
    )(xl, xh, ids3, W_pool, b_pool)


def kernel(x, index_uL, index_vL, attr_uL, batch_ids, W_atom, b_atom,
           We0, be0, WuL0, WvL0, gamma0, beta0,
           We1, be1, WuL1, WvL1, gamma1, beta1,
           We2, be2, WuL2, WvL2, gamma2, beta2,
           W_pool, b_pool):
    i32 = jnp.int32
    f32 = jnp.float32
    pad = EP - E

    def pad_idx(a, val):
        return jnp.concatenate([a, jnp.full((pad,), val, i32)]).reshape(
            NS * NCHUNK, CH)

    dstu = pad_idx(index_uL[0], N)
    srcu = pad_idx(index_uL[1], 0)
    dstv = pad_idx(index_vL[0], N)
    srcv = pad_idx(index_vL[1], 0)
    attr = jnp.concatenate([attr_uL, jnp.zeros((pad, DE), f32)]).T.copy()
    zeros = jnp.zeros((NPAD, H), f32)
    ids3 = batch_ids.reshape(TCGRID, 1, R)

    xl, xh = _atom_call(x, W_atom, b_atom.reshape(1, D))

    sc_layer = _make_sc_layer()
    Wes = jnp.stack([We0, We1, We2])
    bes = jnp.stack([be0, be1, be2])
    Wus = jnp.stack([WuL0, WuL1, WuL2])
    Wvs = jnp.stack([WvL0, WvL1, WvL2])
    gs = jnp.stack([gamma0.reshape(1, D), gamma1.reshape(1, D), gamma2.reshape(1, D)])
    bs = jnp.stack([beta0.reshape(1, D), beta1.reshape(1, D), beta2.reshape(1, D)])

    def step(carry, ws):
        cxl, cxh = carry
        We, be, WuL, WvL, g, b = ws
        aul, auh, avl, avh = sc_layer(
            cxl, cxh, srcu, dstu, srcv, dstv, attr,
            We[:, :H], We[:, H:], be[:H], be[H:], zeros)
        nxl, nxh = _tc_layer_call(aul, auh, avl, avh, cxl, cxh, WuL, WvL, g, b)
        return (nxl, nxh), None

    (xl, xh), _ = lax.scan(step, (xl, xh), (Wes, bes, Wus, Wvs, gs, bs))

    return _pool_call(xl, xh, ids3, W_pool, b_pool.reshape(1, D))
